# baseline (speedup 1.0000x reference)
import functools

import jax
import jax.numpy as jnp
from jax import lax
from jax.experimental import pallas as pl
from jax.experimental.pallas import tpu as pltpu
from jax.experimental.pallas import tpu_sc as plsc

MAX_JUSTICES = 100000
NUM_QUERIES = 8
DIM = 64
BATCH = 16384
ROW = NUM_QUERIES * DIM

NUM_CORES = 2
NUM_SUBCORES = 16
NUM_WORKERS = NUM_CORES * NUM_SUBCORES
IDS_PER_WORKER = BATCH // NUM_WORKERS
CHUNK = 64
NUM_CHUNKS = IDS_PER_WORKER // CHUNK

_mesh = plsc.VectorSubcoreMesh(core_axis_name="c", subcore_axis_name="s")


@functools.partial(
    pl.kernel,
    out_type=jax.ShapeDtypeStruct((BATCH, ROW), jnp.float32),
    mesh=_mesh,
    scratch_types=[
        pltpu.VMEM((NUM_CHUNKS, CHUNK), jnp.int32),
        pltpu.VMEM((2, CHUNK, ROW), jnp.float32),
        pltpu.SemaphoreType.DMA,
        pltpu.SemaphoreType.DMA,
    ],
)
def _gather_kernel(idx_hbm, table_hbm, out_hbm, idx_v, rows_v, sem0, sem1):
    wid = lax.axis_index("s") * NUM_CORES + lax.axis_index("c")
    base = wid * IDS_PER_WORKER

    pltpu.sync_copy(idx_hbm.at[wid], idx_v)

    sems = (sem0, sem1)
    copies = [None, None]
    copies[0] = pltpu.async_copy(table_hbm.at[idx_v.at[0]], rows_v.at[0], sems[0])
    for c in range(NUM_CHUNKS):
        buf = c % 2
        nxt = (c + 1) % 2
        if c + 1 < NUM_CHUNKS:
            copies[nxt] = pltpu.async_copy(
                table_hbm.at[idx_v.at[c + 1]], rows_v.at[nxt], sems[nxt]
            )
        copies[buf].wait()
        pltpu.sync_copy(rows_v.at[buf], out_hbm.at[pl.ds(base + c * CHUNK, CHUNK)])


def kernel(justice_ids, W):
    table = W.reshape(MAX_JUSTICES, ROW)
    idx = justice_ids.astype(jnp.int32).reshape(NUM_WORKERS, NUM_CHUNKS, CHUNK)
    out = _gather_kernel(idx, table)
    return out.reshape(BATCH, NUM_QUERIES, DIM)


# speedup vs baseline: 1.4205x; 1.4205x over previous
import functools

import jax
import jax.numpy as jnp
from jax import lax
from jax.experimental import pallas as pl
from jax.experimental.pallas import tpu as pltpu
from jax.experimental.pallas import tpu_sc as plsc

MAX_JUSTICES = 100000
NUM_QUERIES = 8
DIM = 64
BATCH = 16384

NUM_CORES = 2
NUM_SUBCORES = 16
NUM_WORKERS = NUM_CORES * NUM_SUBCORES
IDS_PER_WORKER = BATCH // NUM_WORKERS
CHUNK = 32
NUM_CHUNKS = IDS_PER_WORKER // CHUNK
LANES = 16

_mesh = plsc.VectorSubcoreMesh(core_axis_name="c", subcore_axis_name="s")


@functools.partial(
    pl.kernel,
    out_type=jax.ShapeDtypeStruct((BATCH, NUM_QUERIES, DIM), jnp.float32),
    mesh=_mesh,
    compiler_params=pltpu.CompilerParams(needs_layout_passes=False),
    scratch_types=[
        pltpu.VMEM((IDS_PER_WORKER,), jnp.int32),
        pltpu.VMEM((2, CHUNK, NUM_QUERIES, DIM), jnp.float32),
        pltpu.SemaphoreType.DMA,
        pltpu.SemaphoreType.DMA,
    ],
)
def _gather_kernel(ids_hbm, table_hbm, out_hbm, ids_v, blocks_v, sem0, sem1):
    wid = lax.axis_index("s") * NUM_CORES + lax.axis_index("c")
    base = wid * IDS_PER_WORKER

    pltpu.sync_copy(ids_hbm.at[pl.ds(base, IDS_PER_WORKER)], ids_v)

    zeros = jnp.zeros((LANES,), jnp.int32)
    sems = (sem0, sem1)

    def start_chunk(c, buf):
        @pl.loop(0, CHUNK)
        def per_id(k):
            idv = plsc.load_gather(ids_v, [zeros + (c * CHUNK + k)])
            row = jnp.max(idv) * NUM_QUERIES
            pltpu.async_copy(
                table_hbm.at[pl.ds(row, NUM_QUERIES)],
                blocks_v.at[buf].at[k],
                sems[buf],
            )

    def wait_chunk(buf):
        @pl.loop(0, CHUNK)
        def per_id(k):
            pltpu.make_async_copy(
                table_hbm.at[pl.ds(0, NUM_QUERIES)],
                blocks_v.at[buf].at[0],
                sems[buf],
            ).wait()

    def drain_chunk(c, buf):
        pltpu.sync_copy(
            blocks_v.at[buf], out_hbm.at[pl.ds(base + c * CHUNK, CHUNK)]
        )

    start_chunk(0, 0)

    @pl.loop(0, NUM_CHUNKS, step=2)
    def ring(c):
        start_chunk(c + 1, 1)
        wait_chunk(0)
        drain_chunk(c, 0)

        @pl.when(c + 2 < NUM_CHUNKS)
        def _():
            start_chunk(c + 2, 0)

        wait_chunk(1)
        drain_chunk(c + 1, 1)


def kernel(justice_ids, W):
    return _gather_kernel(justice_ids.astype(jnp.int32), W)


# speedup vs baseline: 2.0651x; 1.4537x over previous
import functools

import jax
import jax.numpy as jnp
from jax import lax
from jax.experimental import pallas as pl
from jax.experimental.pallas import tpu as pltpu
from jax.experimental.pallas import tpu_sc as plsc

MAX_JUSTICES = 100000
NUM_QUERIES = 8
DIM = 64
BATCH = 16384

NUM_CORES = 2
NUM_SUBCORES = 16
NUM_WORKERS = NUM_CORES * NUM_SUBCORES
IDS_PER_WORKER = BATCH // NUM_WORKERS
CHUNK = 32
NUM_CHUNKS = IDS_PER_WORKER // CHUNK
LANES = 16

_mesh = plsc.VectorSubcoreMesh(core_axis_name="c", subcore_axis_name="s")


@functools.partial(
    pl.kernel,
    out_type=jax.ShapeDtypeStruct((BATCH, NUM_QUERIES, DIM), jnp.float32),
    mesh=_mesh,
    compiler_params=pltpu.CompilerParams(needs_layout_passes=False),
    scratch_types=[
        pltpu.VMEM((IDS_PER_WORKER,), jnp.int32),
        pltpu.VMEM((2, CHUNK, NUM_QUERIES, DIM), jnp.float32),
        pltpu.SemaphoreType.DMA,
        pltpu.SemaphoreType.DMA,
    ],
)
def _gather_kernel(ids_hbm, table_hbm, out_hbm, ids_v, blocks_v, sem0, sem1):
    wid = lax.axis_index("s") * NUM_CORES + lax.axis_index("c")
    base = wid * IDS_PER_WORKER

    pltpu.sync_copy(ids_hbm.at[pl.ds(base, IDS_PER_WORKER)], ids_v)

    lane = lax.iota(jnp.int32, LANES)
    sems = (sem0, sem1)

    def start_chunk(c, buf):
        @pl.loop(0, CHUNK // LANES)
        def per_group(g):
            ids16 = ids_v[pl.ds((c * CHUNK + g * LANES), LANES)]
            for l in range(LANES):
                row = jnp.max(jnp.where(lane == l, ids16, -1))
                pltpu.async_copy(
                    table_hbm.at[row],
                    blocks_v.at[buf].at[g * LANES + l],
                    sems[buf],
                )

    def wait_chunk(buf):
        @pl.loop(0, CHUNK)
        def per_id(k):
            pltpu.make_async_copy(
                table_hbm.at[0],
                blocks_v.at[buf].at[0],
                sems[buf],
            ).wait()

    def drain_chunk(c, buf):
        pltpu.sync_copy(
            blocks_v.at[buf], out_hbm.at[pl.ds(base + c * CHUNK, CHUNK)]
        )

    start_chunk(0, 0)

    @pl.loop(0, NUM_CHUNKS, step=2)
    def ring(c):
        start_chunk(c + 1, 1)
        wait_chunk(0)
        drain_chunk(c, 0)

        @pl.when(c + 2 < NUM_CHUNKS)
        def _():
            start_chunk(c + 2, 0)

        wait_chunk(1)
        drain_chunk(c + 1, 1)


def kernel(justice_ids, W):
    table = W.reshape(MAX_JUSTICES, NUM_QUERIES, DIM)
    return _gather_kernel(justice_ids.astype(jnp.int32), table)


# speedup vs baseline: 2.0721x; 1.0034x over previous
import functools

import jax
import jax.numpy as jnp
from jax import lax
from jax.experimental import pallas as pl
from jax.experimental.pallas import tpu as pltpu
from jax.experimental.pallas import tpu_sc as plsc

MAX_JUSTICES = 100000
NUM_QUERIES = 8
DIM = 64
BATCH = 16384

NUM_CORES = 2
NUM_SUBCORES = 16
NUM_WORKERS = NUM_CORES * NUM_SUBCORES
IDS_PER_WORKER = BATCH // NUM_WORKERS
CHUNK = 32
NUM_CHUNKS = IDS_PER_WORKER // CHUNK
LANES = 16

_mesh = plsc.VectorSubcoreMesh(core_axis_name="c", subcore_axis_name="s")


@functools.partial(
    pl.kernel,
    out_type=jax.ShapeDtypeStruct((BATCH, NUM_QUERIES, DIM), jnp.float32),
    mesh=_mesh,
    compiler_params=pltpu.CompilerParams(needs_layout_passes=False),
    scratch_types=[
        pltpu.VMEM((IDS_PER_WORKER,), jnp.int32),
        pltpu.VMEM((3, CHUNK, NUM_QUERIES, DIM), jnp.float32),
        pltpu.SemaphoreType.DMA,
        pltpu.SemaphoreType.DMA,
        pltpu.SemaphoreType.DMA,
        pltpu.SemaphoreType.DMA,
        pltpu.SemaphoreType.DMA,
        pltpu.SemaphoreType.DMA,
    ],
)
def _gather_kernel(
    ids_hbm, table_hbm, out_hbm, ids_v, blocks_v, g0, g1, g2, d0, d1, d2
):
    wid = lax.axis_index("s") * NUM_CORES + lax.axis_index("c")
    base = wid * IDS_PER_WORKER

    pltpu.sync_copy(ids_hbm.at[pl.ds(base, IDS_PER_WORKER)], ids_v)

    lane = lax.iota(jnp.int32, LANES)
    gsems = (g0, g1, g2)
    dsems = (d0, d1, d2)

    def start_chunk(c, buf):
        @pl.loop(0, CHUNK // LANES)
        def per_group(g):
            ids16 = ids_v[pl.ds((c * CHUNK + g * LANES), LANES)]
            for l in range(LANES):
                row = jnp.max(jnp.where(lane == l, ids16, -1))
                pltpu.async_copy(
                    table_hbm.at[row],
                    blocks_v.at[buf].at[g * LANES + l],
                    gsems[buf],
                )

    def wait_sem(buf, sem):
        pltpu.make_async_copy(
            table_hbm.at[pl.ds(0, CHUNK)], blocks_v.at[buf], sem
        ).wait()

    def stage(c, buf):
        wait_sem(buf, gsems[buf])
        pltpu.async_copy(
            blocks_v.at[buf], out_hbm.at[pl.ds(base + c * CHUNK, CHUNK)], dsems[buf]
        )
        nxt = (buf + 2) % 3

        @pl.when(c + 2 < NUM_CHUNKS)
        def _():
            @pl.when(c >= 1)
            def _():
                wait_sem(nxt, dsems[nxt])

            start_chunk(c + 2, nxt)

    start_chunk(0, 0)
    start_chunk(1, 1)

    @pl.loop(0, NUM_CHUNKS - 1, step=3)
    def ring(c):
        stage(c, 0)
        stage(c + 1, 1)
        stage(c + 2, 2)

    stage(NUM_CHUNKS - 1, 0)
    wait_sem(0, d0)
    wait_sem(1, d1)
    wait_sem(2, d2)


def kernel(justice_ids, W):
    table = W.reshape(MAX_JUSTICES, NUM_QUERIES, DIM)
    return _gather_kernel(justice_ids.astype(jnp.int32), table)
